# Initial kernel scaffold; baseline (speedup 1.0000x reference)
import jax
import jax.numpy as jnp
from jax.experimental import pallas as pl


def kernel(x, Wq, bq, Wk, bk, Wv, bv, Wo, bo, Wqi, bqi, Wki, bki, Wp, bp):
    raise NotImplementedError("write your pallas kernel here")


# speedup vs baseline: 41.2636x; 41.2636x over previous
import jax
import jax.numpy as jnp
import numpy as np
from jax.experimental import pallas as pl

_EMB = 1024
_HEADS = 16
_DK = _EMB // _HEADS
_TOPK = 128
_SCALE = 1.0 / np.sqrt(_DK)
_NEG = -1e30


def _attn_kernel(x_ref, w_all_ref, b_all_ref, wo_ref, bo_ref, out_ref):
    S = x_ref.shape[1]
    xb = x_ref[0]
    f32 = jnp.float32
    bf16 = jnp.bfloat16

    proj = jnp.dot(xb, w_all_ref[...], preferred_element_type=f32)
    proj = proj + b_all_ref[...]

    qi_base = 3 * _EMB
    ki16 = proj[:, 4 * _EMB:4 * _EMB + _DK].astype(bf16)
    wp16 = proj[:, 4 * _EMB + _DK:4 * _EMB + _DK + _HEADS].astype(bf16)

    idx_score = jnp.zeros((S, S), dtype=f32)
    for h in range(_HEADS):
        qih = proj[:, qi_base + h * _DK: qi_base + (h + 1) * _DK].astype(bf16)
        sh = jax.lax.dot_general(qih, ki16, (((1,), (1,)), ((), ())),
                                 preferred_element_type=f32)
        sh16 = jnp.maximum(sh, 0.0).astype(bf16)
        wh = wp16[:, h:h + 1].astype(f32)
        idx_score = idx_score + wh * sh16.astype(f32)

    idx_score = jnp.where(idx_score == 0.0, 0.0, idx_score)

    raw = jax.lax.bitcast_convert_type(idx_score, jnp.int32)
    key = jnp.where(raw >= 0, raw, raw ^ jnp.int32(0x7FFFFFFF))

    i32 = jnp.int32
    posm = jnp.where(key >= 0, i32(1), i32(0))
    cnt_pos = jnp.sum(posm, axis=1, keepdims=True)
    in_pos = jnp.where(cnt_pos >= _TOPK, i32(1), i32(0))
    act = jnp.where(in_pos > 0, posm, 1 - posm)
    kvec = jnp.where(in_pos > 0, i32(_TOPK), i32(_TOPK) - cnt_pos)
    m31 = key & i32(0x7FFFFFFF)
    prefix = jnp.zeros((S, 1), dtype=i32)
    for b in range(30, -1, -1):
        bit = (m31 >> b) & 1
        hit = act * bit
        c1 = jnp.sum(hit, axis=1, keepdims=True)
        take = jnp.where(c1 >= kvec, i32(1), i32(0))
        act = jnp.where(take > 0, hit, act * (1 - bit))
        kvec = jnp.where(take > 0, kvec, kvec - c1)
        prefix = prefix + take * i32(1 << b)

    thr = prefix + i32(-0x80000000) * (1 - in_pos)
    gtm = jnp.where(key > thr, 1.0, 0.0)
    cnt_gt = jnp.sum(gtm, axis=1, keepdims=True)
    need = jnp.float32(_TOPK) - cnt_gt
    eq = jnp.where(key == thr, 1.0, 0.0)
    r = jax.lax.broadcasted_iota(i32, (S, S), 0)
    c = jax.lax.broadcasted_iota(i32, (S, S), 1)
    lt = jnp.where(r <= c, 1.0, 0.0)
    cs = jnp.dot(eq, lt, preferred_element_type=f32)
    take_eq = jnp.where(cs <= need + 0.5, eq, 0.0)
    selv = gtm + take_eq

    neg = (1.0 - selv) * _NEG
    heads = []
    for h in range(_HEADS):
        qh = proj[:, h * _DK:(h + 1) * _DK].astype(bf16)
        kh = proj[:, _EMB + h * _DK:_EMB + (h + 1) * _DK].astype(bf16)
        vh = proj[:, 2 * _EMB + h * _DK:2 * _EMB + (h + 1) * _DK].astype(bf16)
        sh = jax.lax.dot_general(qh, kh, (((1,), (1,)), ((), ())),
                                 preferred_element_type=f32) * _SCALE
        sh = sh + neg
        mx = jnp.max(sh, axis=1, keepdims=True)
        p = jnp.exp(sh - mx)
        denom = jnp.sum(p, axis=1, keepdims=True)
        wts = (p / denom).astype(bf16)
        ah = jnp.dot(wts, vh, preferred_element_type=f32)
        heads.append(ah)
    mh = jnp.concatenate(heads, axis=1).astype(bf16)
    out = jnp.dot(mh, wo_ref[...], preferred_element_type=f32) + bo_ref[...]
    out_ref[0] = out


@jax.jit
def kernel(x, Wq, bq, Wk, bk, Wv, bv, Wo, bo, Wqi, bqi, Wki, bki, Wp, bp):
    B, S, E = x.shape
    bf16 = jnp.bfloat16
    w_all = jnp.concatenate([Wq, Wk, Wv, Wqi, Wki, Wp], axis=1).astype(bf16)
    b_all = jnp.concatenate([bq, bk, bv, bqi, bki, bp])[None, :]
    cols = w_all.shape[1]
    out = pl.pallas_call(
        _attn_kernel,
        grid=(B,),
        in_specs=[
            pl.BlockSpec((1, S, E), lambda b: (b, 0, 0)),
            pl.BlockSpec((E, cols), lambda b: (0, 0)),
            pl.BlockSpec((1, cols), lambda b: (0, 0)),
            pl.BlockSpec((E, E), lambda b: (0, 0)),
            pl.BlockSpec((1, E), lambda b: (0, 0)),
        ],
        out_specs=pl.BlockSpec((1, S, E), lambda b: (b, 0, 0)),
        out_shape=jax.ShapeDtypeStruct((B, S, E), jnp.float32),
    )(x.astype(bf16), w_all, b_all, Wo.astype(bf16), bo[None, :])
    return out


# speedup vs baseline: 65.8122x; 1.5949x over previous
import jax
import jax.numpy as jnp
import numpy as np
from jax.experimental import pallas as pl

_EMB = 1024
_HEADS = 16
_DK = _EMB // _HEADS
_TOPK = 128
_SCALE = 1.0 / np.sqrt(_DK)
_NEG = -1e30


def _attn_kernel(x_ref, wq_ref, wk_ref, wv_ref, wqi_ref, wki_ref, wp_ref,
                 wo_ref, bq_ref, bk_ref, bv_ref, bqi_ref, bki_ref, bp_ref,
                 bo_ref, out_ref):
    S = x_ref.shape[1]
    f32 = jnp.float32
    bf16 = jnp.bfloat16
    xb16 = x_ref[0].astype(bf16)

    def proj(w_ref, b_ref):
        return jnp.dot(xb16, w_ref[...].astype(bf16),
                       preferred_element_type=f32) + b_ref[...]

    qi = proj(wqi_ref, bqi_ref)
    ki16 = proj(wki_ref, bki_ref).astype(bf16)
    wp16 = proj(wp_ref, bp_ref).astype(bf16)

    qi16 = qi.astype(bf16)
    idx_score = jnp.zeros((S, S), dtype=f32)
    for h in range(_HEADS):
        qih = qi16[:, h * _DK:(h + 1) * _DK]
        sh = jax.lax.dot_general(qih, ki16, (((1,), (1,)), ((), ())),
                                 preferred_element_type=f32)
        sh16 = jnp.maximum(sh, 0.0).astype(bf16)
        wh = wp16[:, h:h + 1].astype(f32)
        idx_score = idx_score + wh * sh16.astype(f32)

    idx_score = jnp.where(idx_score == 0.0, 0.0, idx_score)

    raw = jax.lax.bitcast_convert_type(idx_score, jnp.int32)
    key = jnp.where(raw >= 0, raw, raw ^ jnp.int32(0x7FFFFFFF))

    i32 = jnp.int32
    posm = jnp.where(key >= 0, i32(1), i32(0))
    cnt_pos = jnp.sum(posm, axis=1, keepdims=True)
    in_pos = jnp.where(cnt_pos >= _TOPK, i32(1), i32(0))
    act = jnp.where(in_pos > 0, posm, 1 - posm)
    kvec = jnp.where(in_pos > 0, i32(_TOPK), i32(_TOPK) - cnt_pos)
    m31 = key & i32(0x7FFFFFFF)
    prefix = jnp.zeros((S, 1), dtype=i32)
    for b in range(30, -1, -1):
        bit = (m31 >> b) & 1
        hit = act * bit
        c1 = jnp.sum(hit, axis=1, keepdims=True)
        take = jnp.where(c1 >= kvec, i32(1), i32(0))
        act = jnp.where(take > 0, hit, act * (1 - bit))
        kvec = jnp.where(take > 0, kvec, kvec - c1)
        prefix = prefix + take * i32(1 << b)

    thr = prefix + i32(-0x80000000) * (1 - in_pos)
    gtm = jnp.where(key > thr, 1.0, 0.0)
    cnt_gt = jnp.sum(gtm, axis=1, keepdims=True)
    need = jnp.float32(_TOPK) - cnt_gt
    eq = jnp.where(key == thr, 1.0, 0.0)
    r = jax.lax.broadcasted_iota(i32, (S, S), 0)
    c = jax.lax.broadcasted_iota(i32, (S, S), 1)
    lt = jnp.where(r <= c, 1.0, 0.0)
    cs = jnp.dot(eq, lt, preferred_element_type=f32)
    take_eq = jnp.where(cs <= need + 0.5, eq, 0.0)
    selv = gtm + take_eq

    neg = (1.0 - selv) * _NEG
    Q = proj(wq_ref, bq_ref).astype(bf16)
    K = proj(wk_ref, bk_ref).astype(bf16)
    V = proj(wv_ref, bv_ref).astype(bf16)
    heads = []
    for h in range(_HEADS):
        qh = Q[:, h * _DK:(h + 1) * _DK]
        kh = K[:, h * _DK:(h + 1) * _DK]
        vh = V[:, h * _DK:(h + 1) * _DK]
        sh = jax.lax.dot_general(qh, kh, (((1,), (1,)), ((), ())),
                                 preferred_element_type=f32) * _SCALE
        sh = sh + neg
        mx = jnp.max(sh, axis=1, keepdims=True)
        p = jnp.exp(sh - mx)
        denom = jnp.sum(p, axis=1, keepdims=True)
        wts = (p / denom).astype(bf16)
        ah = jnp.dot(wts, vh, preferred_element_type=f32)
        heads.append(ah)
    mh = jnp.concatenate(heads, axis=1).astype(bf16)
    out = jnp.dot(mh, wo_ref[...].astype(bf16),
                  preferred_element_type=f32) + bo_ref[...]
    out_ref[0] = out


@jax.jit
def kernel(x, Wq, bq, Wk, bk, Wv, bv, Wo, bo, Wqi, bqi, Wki, bki, Wp, bp):
    B, S, E = x.shape

    def full(a):
        return pl.BlockSpec(a.shape, lambda b: (0,) * a.ndim)

    biases = [b[None, :] for b in (bq, bk, bv, bqi, bki, bp, bo)]
    weights = [Wq, Wk, Wv, Wqi, Wki, Wp, Wo]
    out = pl.pallas_call(
        _attn_kernel,
        grid=(B,),
        in_specs=[pl.BlockSpec((1, S, E), lambda b: (b, 0, 0))]
        + [full(w) for w in weights]
        + [full(bb) for bb in biases],
        out_specs=pl.BlockSpec((1, S, E), lambda b: (b, 0, 0)),
        out_shape=jax.ShapeDtypeStruct((B, S, E), jnp.float32),
    )(x, *weights, *biases)
    return out
